# Initial kernel scaffold; baseline (speedup 1.0000x reference)
import jax
import jax.numpy as jnp
from jax.experimental import pallas as pl


def kernel(x, src_mask, src_max_len, src_pitch, src_energy, src_duration, mel_mask, max_len, Wd, bd, gd, blnd, Wld, bld, Wp, bp, gp, blnp, Wlp, blp, We, be, ge, blne, Wle, ble, Wp1, bp1, We1, be1):
    raise NotImplementedError("write your pallas kernel here")


# speedup vs baseline: 8.5036x; 8.5036x over previous
import dataclasses
import functools

import jax
import jax.numpy as jnp
from jax import lax
from jax.experimental import pallas as pl
from jax.experimental.pallas import tpu as pltpu
from jax.experimental.pallas import tpu_sc as plsc

_B, _S, _H, _T, _F = 8, 2048, 256, 4096, 256
_NW = 32
_ROWS_PER_W = _B * _T // _NW
_GCHUNK = 128
_ZROW = _B * _S
_LANES = 16


def _var_pred_math(xb, w0, w1, w2, b, g, bln, wl, bl):
    m0 = jnp.dot(xb, w0, preferred_element_type=jnp.float32)
    m1 = jnp.dot(xb, w1, preferred_element_type=jnp.float32)
    m2 = jnp.dot(xb, w2, preferred_element_type=jnp.float32)
    z = jnp.zeros((1, _F), jnp.float32)
    h = m1 + b
    h = h + jnp.concatenate([z, m0[:-1]], axis=0)
    h = h + jnp.concatenate([m2[1:], z], axis=0)
    h = jnp.maximum(h, 0.0)
    mu = jnp.mean(h, axis=1, keepdims=True)
    var = jnp.mean(jnp.square(h - mu), axis=1, keepdims=True)
    hn = (h - mu) * lax.rsqrt(var + 1e-5) * g + bln
    return jnp.sum(hn * wl, axis=1) + bl[0, 0]


def _ka_body(x_ref, p_ref, e_ref, keep_ref, w0, w1, w2, b, g, bln, wl, bl,
             wp1, bp1, we1, be1, x2_ref, ld_ref):
    i = pl.program_id(0)

    @pl.when(i < _B)
    def _():
        xb = x_ref[0]
        pc = p_ref[0, 0][:, None] * wp1[...] + bp1[...]
        ec = e_ref[0, 0][:, None] * we1[...] + be1[...]
        x2_ref[...] = xb + pc + ec
        out = _var_pred_math(xb, w0[...], w1[...], w2[...], b[...], g[...],
                             bln[...], wl[...], bl)
        ld_ref[0, 0, :] = out * keep_ref[0, 0]

    @pl.when(i >= _B)
    def _():
        x2_ref[...] = jnp.zeros_like(x2_ref)
        ld_ref[...] = jnp.zeros_like(ld_ref)


def _kb_body(mel_ref, keep_ref,
             w0p, w1p, w2p, bp_, gp_, blnp_, wlp, blp_,
             w0e, w1e, w2e, be_, ge_, blne_, wle, ble_,
             pp_ref, ep_ref):
    melb = mel_ref[...]
    keep = keep_ref[0, 0]
    pp_ref[0, 0, :] = _var_pred_math(melb, w0p[...], w1p[...], w2p[...],
                                     bp_[...], gp_[...], blnp_[...], wlp[...],
                                     blp_) * keep
    ep_ref[0, 0, :] = _var_pred_math(melb, w0e[...], w1e[...], w2e[...],
                                     be_[...], ge_[...], blne_[...], wle[...],
                                     ble_) * keep


def _sc_idx_body(dur_hbm, flat_hbm, mellen_hbm, d_v, a_v, f_v, ml_v):
    wid = lax.axis_index("s") * 2 + lax.axis_index("c")

    @pl.when(wid < _B)
    def _():
        b = wid
        pltpu.sync_copy(dur_hbm.at[b], d_v)
        iota = lax.iota(jnp.int32, _LANES)
        zeros = jnp.zeros((_LANES,), jnp.int32)

        def init_body(i, carry):
            a_v[pl.ds(i * _LANES, _LANES)] = zeros
            return carry

        lax.fori_loop(0, _T // _LANES, init_body, jnp.int32(0))

        def scat_body(i, tot):
            d = d_v[pl.ds(i * _LANES, _LANES)]
            cs = plsc.cumsum(d) + tot
            ex = cs - d
            mask = (d > 0) & (ex < _T)
            plsc.store_scatter(a_v, [ex], iota + i * _LANES, mask=mask)
            return tot + jnp.sum(d)

        total = lax.fori_loop(0, _S // _LANES, scat_body, jnp.int32(0))
        mel_len = jnp.minimum(total, _T)
        row_base = b * _S

        def cm_body(i, carry):
            ch = jnp.maximum(a_v[pl.ds(i * _LANES, _LANES)], carry)
            mm = plsc.cummax(ch)
            t = iota + i * _LANES
            f_v[pl.ds(i * _LANES, _LANES)] = jnp.where(
                t < mel_len, mm + row_base, _ZROW)
            return jnp.max(mm)

        lax.fori_loop(0, _T // _LANES, cm_body, jnp.int32(0))
        pltpu.sync_copy(f_v, flat_hbm.at[b])
        ml_v[...] = jnp.broadcast_to(mel_len, (_LANES,))
        pltpu.sync_copy(ml_v, mellen_hbm.at[b])


def _sc_gather_body(x2_hbm, flat_hbm, mel_hbm, idx_v, rows_v, sem):
    wid = lax.axis_index("s") * 2 + lax.axis_index("c")
    base = wid * _ROWS_PER_W
    pltpu.sync_copy(flat_hbm.at[pl.ds(base, _ROWS_PER_W)], idx_v)

    def body(j, carry):
        off = j * _GCHUNK
        pltpu.async_copy(
            x2_hbm.at[idx_v.at[pl.ds(off, _GCHUNK)]], rows_v, sem).wait()
        pltpu.sync_copy(rows_v, mel_hbm.at[pl.ds(base + off, _GCHUNK)])
        return carry

    lax.fori_loop(0, _ROWS_PER_W // _GCHUNK, body, jnp.int32(0))


def kernel(x, src_mask, src_max_len, src_pitch, src_energy, src_duration,
           mel_mask, max_len, Wd, bd, gd, blnd, Wld, bld, Wp, bp, gp, blnp,
           Wlp, blp, We, be, ge, blne, Wle, ble, Wp1, bp1, We1, be1):
    f32 = jnp.float32
    keep_s = 1.0 - src_mask.astype(f32)
    keep_m = 1.0 - mel_mask.astype(f32)

    def wrow(v):
        return v.reshape(1, -1).astype(f32)

    full = lambda i: (0, 0)
    batch3 = lambda i: (jnp.minimum(i, _B - 1), 0, 0)

    wspec = pl.BlockSpec((_H, _F), full)
    rspec = pl.BlockSpec((1, _F), full)
    hspec = pl.BlockSpec((1, _H), full)
    sspec = pl.BlockSpec((1, 1), full)
    rowS = pl.BlockSpec((1, 1, _S), batch3)

    x2_ext, logd_pad = pl.pallas_call(
        _ka_body,
        grid=(_B + 1,),
        in_specs=[
            pl.BlockSpec((1, _S, _H), batch3),
            rowS, rowS, rowS,
            wspec, wspec, wspec, rspec, rspec, rspec, rspec, sspec,
            hspec, hspec, hspec, hspec,
        ],
        out_specs=[
            pl.BlockSpec((_S, _H), lambda i: (i, 0)),
            pl.BlockSpec((1, 1, _S), lambda i: (i, 0, 0)),
        ],
        out_shape=[
            jax.ShapeDtypeStruct(((_B + 1) * _S, _H), f32),
            jax.ShapeDtypeStruct((_B + 1, 1, _S), f32),
        ],
    )(x, src_pitch.reshape(_B, 1, _S), src_energy.reshape(_B, 1, _S),
      keep_s.reshape(_B, 1, _S),
      Wd[0], Wd[1], Wd[2], wrow(bd), wrow(gd), wrow(blnd), wrow(Wld),
      bld.reshape(1, 1), Wp1.astype(f32), wrow(bp1), We1.astype(f32),
      wrow(be1))

    mesh = plsc.VectorSubcoreMesh(core_axis_name="c", subcore_axis_name="s")
    sc_params = pltpu.CompilerParams()
    if "needs_layout_passes" in pltpu.CompilerParams.__dataclass_fields__:
        sc_params = dataclasses.replace(sc_params, needs_layout_passes=False)
    flat_idx, mellen16 = pl.kernel(
        _sc_idx_body,
        out_type=[
            jax.ShapeDtypeStruct((_B, _T), jnp.int32),
            jax.ShapeDtypeStruct((_B, _LANES), jnp.int32),
        ],
        mesh=mesh,
        scratch_types=[
            pltpu.VMEM((_S,), jnp.int32),
            pltpu.VMEM((_T,), jnp.int32),
            pltpu.VMEM((_T,), jnp.int32),
            pltpu.VMEM((_LANES,), jnp.int32),
        ],
        compiler_params=sc_params,
    )(src_duration)

    mel2d = pl.kernel(
        _sc_gather_body,
        out_type=jax.ShapeDtypeStruct((_B * _T, _H), f32),
        mesh=mesh,
        scratch_types=[
            pltpu.VMEM((_ROWS_PER_W,), jnp.int32),
            pltpu.VMEM((_GCHUNK, _H), f32),
            pltpu.SemaphoreType.DMA,
        ],
        compiler_params=sc_params,
    )(x2_ext, flat_idx.reshape(_B * _T))

    pitch_pred, energy_pred = pl.pallas_call(
        _kb_body,
        grid=(_B,),
        in_specs=[
            pl.BlockSpec((_T, _H), lambda i: (i, 0)),
            pl.BlockSpec((1, 1, _T), lambda i: (i, 0, 0)),
            wspec, wspec, wspec, rspec, rspec, rspec, rspec, sspec,
            wspec, wspec, wspec, rspec, rspec, rspec, rspec, sspec,
        ],
        out_specs=[
            pl.BlockSpec((1, 1, _T), lambda i: (i, 0, 0)),
            pl.BlockSpec((1, 1, _T), lambda i: (i, 0, 0)),
        ],
        out_shape=[
            jax.ShapeDtypeStruct((_B, 1, _T), f32),
            jax.ShapeDtypeStruct((_B, 1, _T), f32),
        ],
    )(mel2d, keep_m.reshape(_B, 1, _T),
      Wp[0], Wp[1], Wp[2], wrow(bp), wrow(gp), wrow(blnp), wrow(Wlp),
      blp.reshape(1, 1),
      We[0], We[1], We[2], wrow(be), wrow(ge), wrow(blne), wrow(Wle),
      ble.reshape(1, 1))

    mel = mel2d.reshape(_B, _T, _H)
    mel_len = mellen16[:, 0]
    return (mel, logd_pad[:_B].reshape(_B, _S),
            pitch_pred.reshape(_B, _T), energy_pred.reshape(_B, _T), mel_len)
